# baseline (speedup 1.0000x reference)
import functools

import jax
import jax.numpy as jnp
from jax import lax
from jax.experimental import pallas as pl
from jax.experimental.pallas import tpu as pltpu
from jax.experimental.pallas import tpu_sc as plsc

_B = 16384
_D = 128
_NEG = 5
_NC = 2
_NS = 16
_NW = _NC * _NS
_BPW = _B // _NW
_CHUNK = 64
_NCH = _BPW // _CHUNK
_GRP = _CHUNK // 16
_NROWS = _CHUNK * _NEG
_VN = _CHUNK + _NROWS
_LN2 = 0.6931471805599453
_NBUF = 2


def _softplus(z):
  w = 1.0 + jnp.exp(z)
  bits = lax.bitcast_convert_type(w, jnp.int32)
  e = jnp.right_shift(bits, 23) - 127
  m = lax.bitcast_convert_type(
      (bits & 0x7FFFFF) | 0x3F800000, jnp.float32)
  s = (m - 1.0) / (m + 1.0)
  t = s * s
  ln_m = 2.0 * s * (1.0 + t * (1.0 / 3.0 + t * (
      1.0 / 5.0 + t * (1.0 / 7.0 + t * (1.0 / 9.0)))))
  return e.astype(jnp.float32) * _LN2 + ln_m


def _sc_body(pos_u, pos_v, neg_t, node_t, ctx_t, out,
             idx_u, idx_v, idx_n2, idx_n, u0, vn0, u1, vn1, part_v,
             sem0, sem1):
  wid = lax.axis_index("s") * _NC + lax.axis_index("c")
  base = wid * _BPW
  lanes0 = lax.iota(jnp.int32, 16)
  stage = (
      pltpu.async_copy(pos_u.at[pl.ds(base, _BPW)], idx_u, sem0),
      pltpu.async_copy(pos_v.at[pl.ds(base, _BPW)], idx_v, sem0),
      pltpu.async_copy(neg_t.at[pl.ds(0, _NEG), pl.ds(base, _BPW)], idx_n2,
                       sem0),
  )
  for dsc in stage:
    dsc.wait()

  @plsc.parallel_loop(0, _BPW * _NEG // 16, 1, unroll=4)
  def _flatten(j, _=None):
    r = j * 16 + lanes0
    item = r // _NEG
    col = r - item * _NEG
    idx_n[pl.ds(j * 16, 16)] = plsc.load_gather(idx_n2, [col, item])

  bufs = ((u0, vn0), (u1, vn1))
  sems = (sem0, sem1)

  def issue(c):
    ub, vnb = bufs[c % _NBUF]
    sem = sems[c % _NBUF]
    return (
        pltpu.async_copy(node_t.at[idx_u.at[pl.ds(c * _CHUNK, _CHUNK)]], ub, sem),
        pltpu.async_copy(ctx_t.at[idx_v.at[pl.ds(c * _CHUNK, _CHUNK)]],
                         vnb.at[pl.ds(0, _CHUNK)], sem),
        pltpu.async_copy(ctx_t.at[idx_n.at[pl.ds(c * _NROWS, _NROWS)]],
                         vnb.at[pl.ds(_CHUNK, _NROWS)], sem),
    )

  lanes = lax.iota(jnp.int32, 16)
  partial = jnp.zeros((16,), jnp.float32)

  def compute(c, partial):
    ub, vnb = bufs[c % _NBUF]

    def gbody(g, partial):
      rows = lanes + g * 16
      nrows = [rows * _NEG + (_CHUNK + n) for n in range(_NEG)]
      zero = jnp.zeros((16,), jnp.float32)

      @plsc.parallel_loop(0, _D, 1, unroll=16,
                          carry=(zero,) * (1 + _NEG) + (lanes,))
      def acc(d, carry):
        dv = carry[-1]
        u = plsc.load_gather(ub, [rows, dv])
        v = plsc.load_gather(vnb, [rows, dv])
        out_c = [carry[0] + u * v]
        for n in range(_NEG):
          w = plsc.load_gather(vnb, [nrows[n], dv])
          out_c.append(carry[n + 1] + u * w)
        out_c.append((dv + 1) & (_D - 1))
        return tuple(out_c)

      partial = partial + _softplus(-jnp.clip(acc[0], -10.0, 10.0))
      for n in range(_NEG):
        partial = partial + _softplus(jnp.clip(acc[n + 1], -10.0, 10.0))
      return partial

    return lax.fori_loop(0, _GRP, gbody, partial)

  pending = {}
  issued = 0
  for c in range(_NCH):
    while issued < min(c + _NBUF, _NCH):
      pending[issued] = issue(issued)
      issued += 1
    for dsc in pending.pop(c):
      dsc.wait()
    partial = compute(c, partial)

  part_v[...] = partial
  pltpu.sync_copy(part_v, out.at[pl.ds(wid * 16, 16)])


@functools.lru_cache(maxsize=1)
def _sc_dots():
  return pl.kernel(
      _sc_body,
      out_type=jax.ShapeDtypeStruct((_NW * 16,), jnp.float32),
      mesh=plsc.VectorSubcoreMesh(core_axis_name="c", subcore_axis_name="s",
                                  num_cores=_NC, num_subcores=_NS),
      scratch_types=[
          pltpu.VMEM((_BPW,), jnp.int32),
          pltpu.VMEM((_BPW,), jnp.int32),
          pltpu.VMEM((_NEG, _BPW), jnp.int32),
          pltpu.VMEM((_BPW * _NEG,), jnp.int32),
          pltpu.VMEM((_CHUNK, _D), jnp.float32),
          pltpu.VMEM((_VN, _D), jnp.float32),
          pltpu.VMEM((_CHUNK, _D), jnp.float32),
          pltpu.VMEM((_VN, _D), jnp.float32),
          pltpu.VMEM((16,), jnp.float32),
          pltpu.SemaphoreType.DMA,
          pltpu.SemaphoreType.DMA,
      ],
      compiler_params=pltpu.CompilerParams(needs_layout_passes=False),
  )


def kernel(pos_u, pos_v, neg_v, node_embed, context_embed):
  partials = _sc_dots()(pos_u, pos_v, neg_v.T, node_embed, context_embed)
  return jnp.sum(partials) / _B


# speedup vs baseline: 1.0287x; 1.0287x over previous
import functools

import jax
import jax.numpy as jnp
from jax import lax
from jax.experimental import pallas as pl
from jax.experimental.pallas import tpu as pltpu
from jax.experimental.pallas import tpu_sc as plsc

_B = 16384
_D = 128
_NEG = 5
_NC = 2
_NS = 16
_NW = _NC * _NS
_BPW = _B // _NW
_CHUNK = 64
_NCH = _BPW // _CHUNK
_GRP = _CHUNK // 16
_NROWS = _CHUNK * _NEG
_VN = _CHUNK + _NROWS
_LN2 = 0.6931471805599453
_NBUF = 2


def _softplus(z):
  w = 1.0 + jnp.exp(z)
  bits = lax.bitcast_convert_type(w, jnp.int32)
  e = jnp.right_shift(bits, 23) - 127
  m = lax.bitcast_convert_type(
      (bits & 0x7FFFFF) | 0x3F800000, jnp.float32)
  s = (m - 1.0) / (m + 1.0)
  t = s * s
  ln_m = 2.0 * s * (1.0 + t * (1.0 / 3.0 + t * (
      1.0 / 5.0 + t * (1.0 / 7.0 + t * (1.0 / 9.0)))))
  return e.astype(jnp.float32) * _LN2 + ln_m


def _sc_body(pos_u, pos_v, neg_t, node_t, ctx_t, out,
             idx_u, idx_v, idx_n2, idx_n, u0, vn0, u1, vn1, part_v,
             sem0, sem1):
  wid = lax.axis_index("s") * _NC + lax.axis_index("c")
  base = wid * _BPW
  lanes0 = lax.iota(jnp.int32, 16)
  stage = (
      pltpu.async_copy(pos_u.at[pl.ds(base, _BPW)], idx_u, sem0),
      pltpu.async_copy(pos_v.at[pl.ds(base, _BPW)], idx_v, sem0),
      pltpu.async_copy(neg_t.at[pl.ds(0, _NEG), pl.ds(base, _BPW)], idx_n2,
                       sem0),
  )
  for dsc in stage:
    dsc.wait()

  @plsc.parallel_loop(0, _BPW * _NEG // 16, 1, unroll=4)
  def _flatten(j, _=None):
    r = j * 16 + lanes0
    item = r // _NEG
    col = r - item * _NEG
    idx_n[pl.ds(j * 16, 16)] = plsc.load_gather(idx_n2, [col, item])

  bufs = ((u0, vn0), (u1, vn1))
  sems = (sem0, sem1)

  def issue(c):
    ub, vnb = bufs[c % _NBUF]
    sem = sems[c % _NBUF]
    return (
        pltpu.async_copy(node_t.at[idx_u.at[pl.ds(c * _CHUNK, _CHUNK)]], ub, sem),
        pltpu.async_copy(ctx_t.at[idx_v.at[pl.ds(c * _CHUNK, _CHUNK)]],
                         vnb.at[pl.ds(0, _CHUNK)], sem),
        pltpu.async_copy(ctx_t.at[idx_n.at[pl.ds(c * _NROWS, _NROWS)]],
                         vnb.at[pl.ds(_CHUNK, _NROWS)], sem),
    )

  lanes = lax.iota(jnp.int32, 16)
  partial = jnp.zeros((16,), jnp.float32)

  def compute(c, partial):
    ub, vnb = bufs[c % _NBUF]

    def gbody(g, partial):
      rows = lanes + g * 16
      nrows = [rows * _NEG + (_CHUNK + n) for n in range(_NEG)]
      zero = jnp.zeros((16,), jnp.float32)

      @plsc.parallel_loop(0, _D, 1, unroll=4,
                          carry=(zero,) * (1 + _NEG) + (lanes,))
      def acc(d, carry):
        dv = carry[-1]
        u = plsc.load_gather(ub, [rows, dv])
        v = plsc.load_gather(vnb, [rows, dv])
        out_c = [carry[0] + u * v]
        for n in range(_NEG):
          w = plsc.load_gather(vnb, [nrows[n], dv])
          out_c.append(carry[n + 1] + u * w)
        out_c.append((dv + 1) & (_D - 1))
        return tuple(out_c)

      partial = partial + _softplus(-jnp.clip(acc[0], -10.0, 10.0))
      for n in range(_NEG):
        partial = partial + _softplus(jnp.clip(acc[n + 1], -10.0, 10.0))
      return partial

    return lax.fori_loop(0, _GRP, gbody, partial)

  pending = {}
  issued = 0
  for c in range(_NCH):
    while issued < min(c + _NBUF, _NCH):
      pending[issued] = issue(issued)
      issued += 1
    for dsc in pending.pop(c):
      dsc.wait()
    partial = compute(c, partial)

  part_v[...] = partial
  pltpu.sync_copy(part_v, out.at[pl.ds(wid * 16, 16)])


@functools.lru_cache(maxsize=1)
def _sc_dots():
  return pl.kernel(
      _sc_body,
      out_type=jax.ShapeDtypeStruct((_NW * 16,), jnp.float32),
      mesh=plsc.VectorSubcoreMesh(core_axis_name="c", subcore_axis_name="s",
                                  num_cores=_NC, num_subcores=_NS),
      scratch_types=[
          pltpu.VMEM((_BPW,), jnp.int32),
          pltpu.VMEM((_BPW,), jnp.int32),
          pltpu.VMEM((_NEG, _BPW), jnp.int32),
          pltpu.VMEM((_BPW * _NEG,), jnp.int32),
          pltpu.VMEM((_CHUNK, _D), jnp.float32),
          pltpu.VMEM((_VN, _D), jnp.float32),
          pltpu.VMEM((_CHUNK, _D), jnp.float32),
          pltpu.VMEM((_VN, _D), jnp.float32),
          pltpu.VMEM((16,), jnp.float32),
          pltpu.SemaphoreType.DMA,
          pltpu.SemaphoreType.DMA,
      ],
      compiler_params=pltpu.CompilerParams(needs_layout_passes=False),
  )


def kernel(pos_u, pos_v, neg_v, node_embed, context_embed):
  partials = _sc_dots()(pos_u, pos_v, neg_v.T, node_embed, context_embed)
  return jnp.sum(partials) / _B


# speedup vs baseline: 1.0360x; 1.0071x over previous
import functools

import jax
import jax.numpy as jnp
from jax import lax
from jax.experimental import pallas as pl
from jax.experimental.pallas import tpu as pltpu
from jax.experimental.pallas import tpu_sc as plsc

_B = 16384
_D = 128
_NEG = 5
_NC = 2
_NS = 16
_NW = _NC * _NS
_BPW = _B // _NW
_CHUNK = 64
_NCH = _BPW // _CHUNK
_GRP = _CHUNK // 16
_NROWS = _CHUNK * _NEG
_VN = _CHUNK + _NROWS
_LN2 = 0.6931471805599453
_NBUF = 2


def _softplus(z):
  w = 1.0 + jnp.exp(z)
  bits = lax.bitcast_convert_type(w, jnp.int32)
  e = jnp.right_shift(bits, 23) - 127
  m = lax.bitcast_convert_type(
      (bits & 0x7FFFFF) | 0x3F800000, jnp.float32)
  s = (m - 1.0) / (m + 1.0)
  t = s * s
  ln_m = 2.0 * s * (1.0 + t * (1.0 / 3.0 + t * (
      1.0 / 5.0 + t * (1.0 / 7.0 + t * (1.0 / 9.0)))))
  return e.astype(jnp.float32) * _LN2 + ln_m


def _sc_body(pos_u, pos_v, neg_t, node_t, ctx_t, out,
             idx_u, idx_v, idx_n2, idx_n, u0, vn0, u1, vn1, part_v,
             sem0, sem1):
  wid = lax.axis_index("s") * _NC + lax.axis_index("c")
  base = wid * _BPW
  lanes0 = lax.iota(jnp.int32, 16)
  stage = (
      pltpu.async_copy(pos_u.at[pl.ds(base, _BPW)], idx_u, sem0),
      pltpu.async_copy(pos_v.at[pl.ds(base, _BPW)], idx_v, sem0),
      pltpu.async_copy(neg_t.at[pl.ds(0, _NEG), pl.ds(base, _BPW)], idx_n2,
                       sem0),
  )
  for dsc in stage:
    dsc.wait()

  @plsc.parallel_loop(0, _BPW * _NEG // 16, 1, unroll=4)
  def _flatten(j, _=None):
    r = j * 16 + lanes0
    item = r // _NEG
    col = r - item * _NEG
    idx_n[pl.ds(j * 16, 16)] = plsc.load_gather(idx_n2, [col, item])

  bufs = ((u0, vn0), (u1, vn1))
  sems = (sem0, sem1)

  def issue(c):
    ub, vnb = bufs[c % _NBUF]
    sem = sems[c % _NBUF]
    return (
        pltpu.async_copy(node_t.at[idx_u.at[pl.ds(c * _CHUNK, _CHUNK)]], ub, sem),
        pltpu.async_copy(ctx_t.at[idx_v.at[pl.ds(c * _CHUNK, _CHUNK)]],
                         vnb.at[pl.ds(0, _CHUNK)], sem),
        pltpu.async_copy(ctx_t.at[idx_n.at[pl.ds(c * _NROWS, _NROWS)]],
                         vnb.at[pl.ds(_CHUNK, _NROWS)], sem),
    )

  lanes = lax.iota(jnp.int32, 16)
  partial = jnp.zeros((16,), jnp.float32)

  def compute(c, partial):
    ub, vnb = bufs[c % _NBUF]

    def gbody(g, partial):
      rows = lanes + g * 16
      nrows = [rows * _NEG + (_CHUNK + n) for n in range(_NEG)]
      zero = jnp.zeros((16,), jnp.float32)

      @plsc.parallel_loop(0, _D, 1, unroll=2,
                          carry=(zero,) * (1 + _NEG) + (lanes,))
      def acc(d, carry):
        dv = carry[-1]
        u = plsc.load_gather(ub, [rows, dv])
        v = plsc.load_gather(vnb, [rows, dv])
        out_c = [carry[0] + u * v]
        for n in range(_NEG):
          w = plsc.load_gather(vnb, [nrows[n], dv])
          out_c.append(carry[n + 1] + u * w)
        out_c.append((dv + 1) & (_D - 1))
        return tuple(out_c)

      partial = partial + _softplus(-jnp.clip(acc[0], -10.0, 10.0))
      for n in range(_NEG):
        partial = partial + _softplus(jnp.clip(acc[n + 1], -10.0, 10.0))
      return partial

    return lax.fori_loop(0, _GRP, gbody, partial)

  pending = {}
  issued = 0
  for c in range(_NCH):
    while issued < min(c + _NBUF, _NCH):
      pending[issued] = issue(issued)
      issued += 1
    for dsc in pending.pop(c):
      dsc.wait()
    partial = compute(c, partial)

  part_v[...] = partial
  pltpu.sync_copy(part_v, out.at[pl.ds(wid * 16, 16)])


@functools.lru_cache(maxsize=1)
def _sc_dots():
  return pl.kernel(
      _sc_body,
      out_type=jax.ShapeDtypeStruct((_NW * 16,), jnp.float32),
      mesh=plsc.VectorSubcoreMesh(core_axis_name="c", subcore_axis_name="s",
                                  num_cores=_NC, num_subcores=_NS),
      scratch_types=[
          pltpu.VMEM((_BPW,), jnp.int32),
          pltpu.VMEM((_BPW,), jnp.int32),
          pltpu.VMEM((_NEG, _BPW), jnp.int32),
          pltpu.VMEM((_BPW * _NEG,), jnp.int32),
          pltpu.VMEM((_CHUNK, _D), jnp.float32),
          pltpu.VMEM((_VN, _D), jnp.float32),
          pltpu.VMEM((_CHUNK, _D), jnp.float32),
          pltpu.VMEM((_VN, _D), jnp.float32),
          pltpu.VMEM((16,), jnp.float32),
          pltpu.SemaphoreType.DMA,
          pltpu.SemaphoreType.DMA,
      ],
      compiler_params=pltpu.CompilerParams(needs_layout_passes=False),
  )


def kernel(pos_u, pos_v, neg_v, node_embed, context_embed):
  partials = _sc_dots()(pos_u, pos_v, neg_v.T, node_embed, context_embed)
  return jnp.sum(partials) / _B
